# baseline (speedup 1.0000x reference)
import jax
import jax.numpy as jnp
from jax import lax
from jax.experimental import pallas as pl
from jax.experimental.pallas import tpu as pltpu
from jax.experimental.pallas import tpu_sc as plsc

N_NODES = 10000
N_EDGES = 160000
NPAD = 10240
EPAD = 163840
IN_DIM = 256
HID = 256
HALF = 128
OUT_DIM = 64
N_TILES = 16
CHUNKS = 80
CHUNK = 128
NSEG = NPAD // N_TILES
ROWQ = NSEG // CHUNK
NROWS = NPAD // CHUNK
NHALF = NPAD // 2
TRASH = NHALF
AROWS = 5248


def _mm_body(x_ref, w0_ref, b0_ref, a0_ref, w1_ref, b1_ref, a1_ref,
             wh0_ref, wh1_ref, s0_ref, s1_ref):
    xb = x_ref[...]
    for w_ref, b_ref, a_ref, wh_ref, s_ref in (
            (w0_ref, b0_ref, a0_ref, wh0_ref, s0_ref),
            (w1_ref, b1_ref, a1_ref, wh1_ref, s1_ref)):
        wh = jnp.dot(xb, w_ref[...], preferred_element_type=jnp.float32)
        wh = wh + b_ref[...]
        wh_ref[...] = jnp.stack([wh[:, :HALF], wh[:, HALF:]], axis=0)
        sv = lax.dot_general(wh, a_ref[...], (((1,), (1,)), ((), ())),
                             preferred_element_type=jnp.float32)
        s_ref[...] = sv.T[None]


def _matmuls(xp, W0, b0r, a0r, W1, b1r, a1r):
    n_blk = NPAD // 256
    full = lambda shape: pl.BlockSpec(shape, lambda i: (0,) * len(shape))
    return pl.pallas_call(
        _mm_body,
        grid=(n_blk,),
        in_specs=[
            pl.BlockSpec((256, IN_DIM), lambda i: (i, 0)),
            full((IN_DIM, HID)), full((1, HID)), full((2, HID)),
            full((IN_DIM, HID)), full((1, HID)), full((2, HID)),
        ],
        out_specs=[
            pl.BlockSpec((2, 256, HALF), lambda i: (0, i, 0)),
            pl.BlockSpec((2, 256, HALF), lambda i: (0, i, 0)),
            pl.BlockSpec((1, 2, 256), lambda i: (i, 0, 0)),
            pl.BlockSpec((1, 2, 256), lambda i: (i, 0, 0)),
        ],
        out_shape=[
            jax.ShapeDtypeStruct((2, NPAD, HALF), jnp.float32),
            jax.ShapeDtypeStruct((2, NPAD, HALF), jnp.float32),
            jax.ShapeDtypeStruct((n_blk, 2, 256), jnp.float32),
            jax.ShapeDtypeStruct((n_blk, 2, 256), jnp.float32),
        ],
    )(xp, W0, b0r, a0r, W1, b1r, a1r)


def _sc_body(tab0, tab1, ss0, sd0, ss1, sd1, es0, ed0, es1, ed1, z,
             hg_out,
             sbuf_v, rowsb_v, src_v, dst_v, ex_v,
             dloc_v, dstpa_v, dstpb_v,
             den0q_v, den1q_v, degq_v, hgq_v, iota80_v,
             sema, semb, sema2, semb2,
             a_sh, den0_sh, den1_sh, deg_sh):
    c = lax.axis_index("c")
    s = lax.axis_index("s")
    row0 = s * NSEG

    for j in range(NROWS // 16):
        iota80_v[pl.ds(j * 16, 16)] = lax.iota(jnp.int32, 16) + j * 16

    pltpu.sync_copy(z.at[pl.ds(0, NROWS)], dloc_v)
    sl_seg = pl.ds(s * ROWQ, ROWQ)
    pltpu.sync_copy(dloc_v.at[sl_seg], den0_sh.at[sl_seg])
    pltpu.sync_copy(dloc_v.at[sl_seg], den1_sh.at[sl_seg])
    pltpu.sync_copy(dloc_v.at[sl_seg], deg_sh.at[sl_seg])
    plsc.subcore_barrier()

    def phase1(es_hbm, ed_hbm, ss_hbm, sd_hbm, histo):
        pltpu.sync_copy(ss_hbm, sbuf_v.at[pl.ds(0, NROWS)])
        pltpu.sync_copy(sd_hbm, sbuf_v.at[pl.ds(NROWS, NROWS)])
        pltpu.sync_copy(es_hbm.at[s], src_v)
        pltpu.sync_copy(ed_hbm.at[s], dst_v)

        def outer(k, _):
            for j in range(8):
                sl = pl.ds(j * 16, 16)
                sidx = src_v[k, sl]
                didx = dst_v[k, sl]
                shi = lax.shift_right_logical(sidx, 7)
                slo = jnp.bitwise_and(sidx, 127)
                dhi = lax.shift_right_logical(didx, 7)
                dlo = jnp.bitwise_and(didx, 127)
                sv = plsc.load_gather(sbuf_v, [shi, slo])
                dv = plsc.load_gather(sbuf_v, [dhi + NROWS, dlo])
                e = sv + dv
                e = jnp.where(e > 0.0, e, 0.01 * e)
                ex = jnp.exp(e)
                ex_v[k, sl] = ex
                if histo:
                    plsc.addupdate_scatter(dloc_v, [dhi, dlo], ex)
                src_v[k, sl] = sidx + c * NPAD
            return 0

        lax.fori_loop(0, CHUNKS, outer, 0)

    phase1(es0, ed0, ss0, sd0, True)
    pltpu.sync_copy(dloc_v, den0_sh.at[iota80_v], add=True)
    pltpu.sync_copy(z.at[pl.ds(0, NROWS)], dloc_v)
    phase1(es1, ed1, ss1, sd1, True)
    pltpu.sync_copy(dloc_v, den1_sh.at[iota80_v], add=True)
    pltpu.sync_copy(z.at[pl.ds(0, NROWS)], dloc_v)

    for es_hbm in (es0, es1):
        pltpu.sync_copy(es_hbm.at[s], src_v)

        def dcount(k, _):
            for j in range(8):
                sl = pl.ds(j * 16, 16)
                sidx = src_v[k, sl]
                shi = lax.shift_right_logical(sidx, 7)
                slo = jnp.bitwise_and(sidx, 127)
                plsc.addupdate_scatter(dloc_v, [shi, slo],
                                       jnp.ones((16,), jnp.float32))
            return 0

        lax.fori_loop(0, CHUNKS, dcount, 0)
    pltpu.sync_copy(dloc_v, deg_sh.at[iota80_v], add=True)
    plsc.subcore_barrier()

    def fpass(p, _):
        lo = p * NHALF

        bufa = sbuf_v.at[pl.ds(0, CHUNK)]

        def phase2(tab_hbm):
            def do_chunk(k, buf_dma, gsem, ssem, buf_ref, dstp_ref):
                pltpu.make_async_copy(tab_hbm.at[src_v.at[k]], buf_dma,
                                      gsem).wait()

                def inner(g, _):
                    exv16 = ex_v[k, pl.ds(pl.multiple_of(g * 16, 16), 16)]
                    for t in range(16):
                        r = g * 16 + t
                        exb = jnp.full((16,), exv16[t], jnp.float32)
                        for j in range(HALF // 16):
                            sl = pl.ds(j * 16, 16)
                            buf_ref[r, sl] = buf_ref[r, sl] * exb
                    return 0

                lax.fori_loop(0, 8, inner, 0)
                for j in range(8):
                    sl = pl.ds(j * 16, 16)
                    dd = dst_v[k, sl] - lo
                    ok = jnp.logical_and(dd >= 0, dd < NHALF)
                    dstp_ref[sl] = jnp.where(ok, dd, TRASH)
                pltpu.async_copy(buf_dma, a_sh.at[dstp_ref], ssem, add=True)

            def drain_scatter(buf_dma, ssem, dstp_ref):
                pltpu.make_async_copy(buf_dma, a_sh.at[dstp_ref], ssem).wait()

            pltpu.async_copy(tab_hbm.at[src_v.at[0]], bufa, sema)
            pltpu.async_copy(tab_hbm.at[src_v.at[1]], rowsb_v, semb)

            def outer(kk, _):
                k = 2 * kk
                do_chunk(k, bufa, sema, sema2, sbuf_v, dstpa_v)
                do_chunk(k + 1, rowsb_v, semb, semb2, rowsb_v, dstpb_v)

                @pl.when(kk < CHUNKS // 2 - 1)
                def _():
                    drain_scatter(bufa, sema2, dstpa_v)
                    pltpu.async_copy(tab_hbm.at[src_v.at[k + 2]], bufa, sema)
                    drain_scatter(rowsb_v, semb2, dstpb_v)
                    pltpu.async_copy(tab_hbm.at[src_v.at[k + 3]], rowsb_v,
                                     semb)

                return 0

            lax.fori_loop(0, CHUNKS // 2, outer, 0)
            drain_scatter(bufa, sema2, dstpa_v)
            drain_scatter(rowsb_v, semb2, dstpb_v)

        def p3a(q, _):
            l0 = row0 - lo + q * CHUNK
            rr = s * ROWQ + q
            frows = sbuf_v.at[pl.ds(0, CHUNK)]
            pltpu.sync_copy(a_sh.at[pl.ds(l0, CHUNK)], frows)
            pltpu.sync_copy(den0_sh.at[rr], den0q_v)
            pltpu.sync_copy(den1_sh.at[rr], den1q_v)

            def inner(g, _):
                sl16 = pl.ds(pl.multiple_of(g * 16, 16), 16)
                d0v = den0q_v[sl16]
                d1v = den1q_v[sl16]
                d0v = jnp.where(d0v > 0.0, d0v, 1.0)
                d1v = jnp.where(d1v > 0.0, d1v, 1.0)
                rv16 = d1v / d0v
                for t in range(16):
                    r = g * 16 + t
                    rb = jnp.full((16,), rv16[t], jnp.float32)
                    for j in range(HALF // 16):
                        sl = pl.ds(j * 16, 16)
                        sbuf_v[r, sl] = sbuf_v[r, sl] * rb
                return 0

            lax.fori_loop(0, 8, inner, 0)
            pltpu.sync_copy(frows, a_sh.at[pl.ds(l0, CHUNK)])
            return 0

        def p3b(q, _):
            l0 = row0 - lo + q * CHUNK
            rr = s * ROWQ + q
            frows = sbuf_v.at[pl.ds(0, CHUNK)]
            pltpu.sync_copy(a_sh.at[pl.ds(l0, CHUNK)], frows)
            pltpu.sync_copy(den1_sh.at[rr], den1q_v)
            pltpu.sync_copy(deg_sh.at[rr], degq_v)

            def inner(g, _):
                sl16 = pl.ds(pl.multiple_of(g * 16, 16), 16)
                d1v = den1q_v[sl16]
                d1v = jnp.where(d1v > 0.0, d1v, 1.0)
                inv16 = 1.0 / d1v
                nodev = row0 + q * CHUNK + g * 16 + lax.iota(jnp.int32, 16)
                w16 = jnp.where(nodev < N_NODES, degq_v[sl16], 0.0)
                for t in range(16):
                    r = g * 16 + t
                    invb = jnp.full((16,), inv16[t], jnp.float32)
                    wb = jnp.full((16,), w16[t], jnp.float32)
                    for j in range(HALF // 16):
                        sl = pl.ds(j * 16, 16)
                        hv = sbuf_v[r, sl] * invb
                        hv = jnp.where(hv > 0.0, hv, jnp.exp(hv) - 1.0)
                        hgq_v[sl] = hgq_v[sl] + wb * hv
                return 0

            lax.fori_loop(0, 8, inner, 0)
            return 0

        for i in range(3):
            cid = 3 * s + i

            @pl.when(cid < NHALF // CHUNK)
            def _():
                pltpu.sync_copy(z, a_sh.at[pl.ds(cid * CHUNK, CHUNK)])
        for j in range(HALF // 16):
            hgq_v[pl.ds(j * 16, 16)] = jnp.zeros((16,), jnp.float32)
        plsc.subcore_barrier()
        phase1(es0, ed0, ss0, sd0, False)
        phase2(tab0)
        plsc.subcore_barrier()
        own = (s // 8) == p

        @pl.when(own)
        def _():
            lax.fori_loop(0, ROWQ, p3a, 0)

        plsc.subcore_barrier()
        phase1(es1, ed1, ss1, sd1, False)
        phase2(tab1)
        plsc.subcore_barrier()

        @pl.when(own)
        def _():
            lax.fori_loop(0, ROWQ, p3b, 0)

        wid = p * (2 * N_TILES) + c * N_TILES + s
        pltpu.sync_copy(hgq_v, hg_out.at[wid])
        plsc.subcore_barrier()
        return 0

    lax.fori_loop(0, 2, fpass, 0)


def _sc_call(tab0, tab1, ss0, sd0, ss1, sd1, es0, ed0, es1, ed1, z):
    mesh = plsc.VectorSubcoreMesh(core_axis_name="c", subcore_axis_name="s")
    f32 = jnp.float32
    i32 = jnp.int32
    kern = pl.kernel(
        _sc_body,
        out_type=jax.ShapeDtypeStruct((4 * N_TILES, HALF), f32),
        mesh=mesh,
        compiler_params=pltpu.CompilerParams(needs_layout_passes=False),
        scratch_types=[
            pltpu.VMEM((2 * NROWS, CHUNK), f32),
            pltpu.VMEM((CHUNK, HALF), f32),
            pltpu.VMEM((CHUNKS, CHUNK), i32),
            pltpu.VMEM((CHUNKS, CHUNK), i32),
            pltpu.VMEM((CHUNKS, CHUNK), f32),
            pltpu.VMEM((NROWS, CHUNK), f32),
            pltpu.VMEM((CHUNK,), i32),
            pltpu.VMEM((CHUNK,), i32),
            pltpu.VMEM((CHUNK,), f32),
            pltpu.VMEM((CHUNK,), f32),
            pltpu.VMEM((CHUNK,), f32),
            pltpu.VMEM((HALF,), f32),
            pltpu.VMEM((NROWS,), i32),
            pltpu.SemaphoreType.DMA,
            pltpu.SemaphoreType.DMA,
            pltpu.SemaphoreType.DMA,
            pltpu.SemaphoreType.DMA,
            pltpu.VMEM_SHARED((AROWS, HALF), f32),
            pltpu.VMEM_SHARED((NROWS, CHUNK), f32),
            pltpu.VMEM_SHARED((NROWS, CHUNK), f32),
            pltpu.VMEM_SHARED((NROWS, CHUNK), f32),
        ],
    )
    return kern(tab0, tab1, ss0, sd0, ss1, sd1, es0, ed0, es1, ed1, z)


def _fin_body(parts_ref, wout_ref, bout_ref, out_ref):
    p = parts_ref[...]
    lo = jnp.sum(p[:N_TILES], axis=0) + jnp.sum(p[2 * N_TILES:3 * N_TILES],
                                                axis=0)
    hi = jnp.sum(p[N_TILES:2 * N_TILES], axis=0) + jnp.sum(p[3 * N_TILES:],
                                                           axis=0)
    hg = jnp.concatenate([lo, hi]) * jnp.float32(1.0 / N_NODES)
    r = jnp.dot(hg[None, :], wout_ref[...], preferred_element_type=jnp.float32)
    out_ref[...] = jax.nn.sigmoid(r + bout_ref[...])


def _final(parts, Wout, boutr):
    return pl.pallas_call(
        _fin_body,
        out_shape=jax.ShapeDtypeStruct((1, OUT_DIM), jnp.float32),
    )(parts, Wout, boutr)


def kernel(x, edge_index_rel0, edge_index_rel1, W0, b0, a0, W1, b1, a1,
           Wout, bout):
    f32 = jnp.float32
    xp = jnp.pad(x, ((0, NPAD - N_NODES), (0, 0)))
    b0r = b0.reshape(1, HID)
    b1r = b1.reshape(1, HID)
    a0r = a0[:, 0].reshape(2, HID)
    a1r = a1[:, 0].reshape(2, HID)

    wh0, wh1, s0, s1 = _matmuls(xp, W0, b0r, a0r, W1, b1r, a1r)
    tab0 = wh0.reshape(2 * NPAD, HALF)
    tab1 = wh1.reshape(2 * NPAD, HALF)
    ss0 = s0[:, 0, :].reshape(NROWS, CHUNK)
    sd0 = s0[:, 1, :].reshape(NROWS, CHUNK)
    ss1 = s1[:, 0, :].reshape(NROWS, CHUNK)
    sd1 = s1[:, 1, :].reshape(NROWS, CHUNK)

    pad = jnp.full((EPAD - N_EDGES,), N_NODES, jnp.int32)
    es0 = jnp.concatenate([edge_index_rel0[0], pad]).reshape(N_TILES, CHUNKS, CHUNK)
    ed0 = jnp.concatenate([edge_index_rel0[1], pad]).reshape(N_TILES, CHUNKS, CHUNK)
    es1 = jnp.concatenate([edge_index_rel1[0], pad]).reshape(N_TILES, CHUNKS, CHUNK)
    ed1 = jnp.concatenate([edge_index_rel1[1], pad]).reshape(N_TILES, CHUNKS, CHUNK)

    z = jnp.zeros((CHUNK, CHUNK), f32)

    parts = _sc_call(tab0, tab1, ss0, sd0, ss1, sd1, es0, ed0, es1, ed1, z)
    return _final(parts, Wout, bout.reshape(1, OUT_DIM))


# speedup vs baseline: 1.0445x; 1.0445x over previous
import jax
import jax.numpy as jnp
from jax import lax
from jax.experimental import pallas as pl
from jax.experimental.pallas import tpu as pltpu
from jax.experimental.pallas import tpu_sc as plsc

N_NODES = 10000
N_EDGES = 160000
NPAD = 10240
EPAD = 163840
IN_DIM = 256
HID = 256
HALF = 128
OUT_DIM = 64
N_TILES = 16
CHUNKS = 80
CHUNK = 128
NSEG = NPAD // N_TILES
ROWQ = NSEG // CHUNK
NROWS = NPAD // CHUNK
NHALF = NPAD // 2
TRASH = NHALF
AROWS = 5248


def _mm_body(x_ref, w0_ref, b0_ref, a0_ref, w1_ref, b1_ref, a1_ref,
             wh0_ref, wh1_ref, s0_ref, s1_ref):
    xb = x_ref[...]
    for w_ref, b_ref, a_ref, wh_ref, s_ref in (
            (w0_ref, b0_ref, a0_ref, wh0_ref, s0_ref),
            (w1_ref, b1_ref, a1_ref, wh1_ref, s1_ref)):
        wh = jnp.dot(xb, w_ref[...], preferred_element_type=jnp.float32)
        wh = wh + b_ref[...]
        wh_ref[...] = jnp.stack([wh[:, :HALF], wh[:, HALF:]], axis=0)
        sv = lax.dot_general(wh, a_ref[...], (((1,), (1,)), ((), ())),
                             preferred_element_type=jnp.float32)
        s_ref[...] = sv.T[None]


def _matmuls(xp, W0, b0r, a0r, W1, b1r, a1r):
    n_blk = NPAD // 256
    full = lambda shape: pl.BlockSpec(shape, lambda i: (0,) * len(shape))
    return pl.pallas_call(
        _mm_body,
        grid=(n_blk,),
        in_specs=[
            pl.BlockSpec((256, IN_DIM), lambda i: (i, 0)),
            full((IN_DIM, HID)), full((1, HID)), full((2, HID)),
            full((IN_DIM, HID)), full((1, HID)), full((2, HID)),
        ],
        out_specs=[
            pl.BlockSpec((2, 256, HALF), lambda i: (0, i, 0)),
            pl.BlockSpec((2, 256, HALF), lambda i: (0, i, 0)),
            pl.BlockSpec((1, 2, 256), lambda i: (i, 0, 0)),
            pl.BlockSpec((1, 2, 256), lambda i: (i, 0, 0)),
        ],
        out_shape=[
            jax.ShapeDtypeStruct((2, NPAD, HALF), jnp.float32),
            jax.ShapeDtypeStruct((2, NPAD, HALF), jnp.float32),
            jax.ShapeDtypeStruct((n_blk, 2, 256), jnp.float32),
            jax.ShapeDtypeStruct((n_blk, 2, 256), jnp.float32),
        ],
    )(xp, W0, b0r, a0r, W1, b1r, a1r)


def _sc_body(tab0, tab1, ss0, sd0, ss1, sd1, es0, ed0, es1, ed1, z,
             hg_out,
             sbuf_v, rowsb_v, src_v, dst_v, ex_v,
             dloc_v, dstpa_v, dstpb_v,
             den0q_v, den1q_v, degq_v, hgq_v, iota80_v,
             sema, semb, sema2, semb2,
             a_sh, den0_sh, den1_sh, deg_sh):
    c = lax.axis_index("c")
    s = lax.axis_index("s")
    row0 = s * NSEG

    for j in range(NROWS // 16):
        iota80_v[pl.ds(j * 16, 16)] = lax.iota(jnp.int32, 16) + j * 16

    pltpu.sync_copy(z.at[pl.ds(0, NROWS)], dloc_v)
    sl_seg = pl.ds(s * ROWQ, ROWQ)
    pltpu.sync_copy(dloc_v.at[sl_seg], den0_sh.at[sl_seg])
    pltpu.sync_copy(dloc_v.at[sl_seg], den1_sh.at[sl_seg])
    pltpu.sync_copy(dloc_v.at[sl_seg], deg_sh.at[sl_seg])
    plsc.subcore_barrier()

    def phase1(es_hbm, ed_hbm, ss_hbm, sd_hbm, histo):
        pltpu.sync_copy(ss_hbm, sbuf_v.at[pl.ds(0, NROWS)])
        pltpu.sync_copy(sd_hbm, sbuf_v.at[pl.ds(NROWS, NROWS)])
        pltpu.sync_copy(es_hbm.at[s], src_v)
        pltpu.sync_copy(ed_hbm.at[s], dst_v)

        def outer(k, _):
            for j in range(8):
                sl = pl.ds(j * 16, 16)
                sidx = src_v[k, sl]
                didx = dst_v[k, sl]
                shi = lax.shift_right_logical(sidx, 7)
                slo = jnp.bitwise_and(sidx, 127)
                dhi = lax.shift_right_logical(didx, 7)
                dlo = jnp.bitwise_and(didx, 127)
                sv = plsc.load_gather(sbuf_v, [shi, slo])
                dv = plsc.load_gather(sbuf_v, [dhi + NROWS, dlo])
                e = sv + dv
                e = jnp.where(e > 0.0, e, 0.01 * e)
                ex = jnp.exp(e)
                ex_v[k, sl] = ex
                if histo:
                    plsc.addupdate_scatter(dloc_v, [dhi, dlo], ex)
                src_v[k, sl] = sidx + c * NPAD
            return 0

        lax.fori_loop(0, CHUNKS, outer, 0)

    phase1(es0, ed0, ss0, sd0, True)
    pltpu.sync_copy(dloc_v, den0_sh.at[iota80_v], add=True)
    pltpu.sync_copy(z.at[pl.ds(0, NROWS)], dloc_v)
    phase1(es1, ed1, ss1, sd1, True)
    pltpu.sync_copy(dloc_v, den1_sh.at[iota80_v], add=True)
    pltpu.sync_copy(z.at[pl.ds(0, NROWS)], dloc_v)

    for es_hbm in (es0, es1):
        pltpu.sync_copy(es_hbm.at[s], src_v)

        def dcount(k, _):
            for j in range(8):
                sl = pl.ds(j * 16, 16)
                sidx = src_v[k, sl]
                shi = lax.shift_right_logical(sidx, 7)
                slo = jnp.bitwise_and(sidx, 127)
                plsc.addupdate_scatter(dloc_v, [shi, slo],
                                       jnp.ones((16,), jnp.float32))
            return 0

        lax.fori_loop(0, CHUNKS, dcount, 0)
    pltpu.sync_copy(dloc_v, deg_sh.at[iota80_v], add=True)
    plsc.subcore_barrier()

    def fpass(p, _):
        lo = p * NHALF

        bufa = sbuf_v.at[pl.ds(0, CHUNK)]

        def phase2(tab_hbm):
            def do_chunk(k, buf_dma, sem, buf_ref, dstp_ref):
                pltpu.make_async_copy(tab_hbm.at[src_v.at[k]], buf_dma,
                                      sem).wait()

                def inner(g, _):
                    exv16 = ex_v[k, pl.ds(pl.multiple_of(g * 16, 16), 16)]
                    for t in range(16):
                        r = g * 16 + t
                        exb = jnp.full((16,), exv16[t], jnp.float32)
                        for j in range(HALF // 16):
                            sl = pl.ds(j * 16, 16)
                            buf_ref[r, sl] = buf_ref[r, sl] * exb
                    return 0

                lax.fori_loop(0, 8, inner, 0)
                for j in range(8):
                    sl = pl.ds(j * 16, 16)
                    dd = dst_v[k, sl] - lo
                    ok = jnp.logical_and(dd >= 0, dd < NHALF)
                    dstp_ref[sl] = jnp.where(ok, dd, TRASH)
                pltpu.sync_copy(buf_dma, a_sh.at[dstp_ref], add=True)

            pltpu.async_copy(tab_hbm.at[src_v.at[0]], bufa, sema)

            def outer(kk, _):
                k = 2 * kk
                pltpu.async_copy(tab_hbm.at[src_v.at[k + 1]], rowsb_v, semb)
                do_chunk(k, bufa, sema, sbuf_v, dstpa_v)

                @pl.when(kk < CHUNKS // 2 - 1)
                def _():
                    pltpu.async_copy(tab_hbm.at[src_v.at[k + 2]], bufa, sema)

                do_chunk(k + 1, rowsb_v, semb, rowsb_v, dstpb_v)
                return 0

            lax.fori_loop(0, CHUNKS // 2, outer, 0)

        def p3a(q, _):
            l0 = row0 - lo + q * CHUNK
            rr = s * ROWQ + q
            frows = sbuf_v.at[pl.ds(0, CHUNK)]
            pltpu.sync_copy(a_sh.at[pl.ds(l0, CHUNK)], frows)
            pltpu.sync_copy(den0_sh.at[rr], den0q_v)
            pltpu.sync_copy(den1_sh.at[rr], den1q_v)

            def inner(g, _):
                sl16 = pl.ds(pl.multiple_of(g * 16, 16), 16)
                d0v = den0q_v[sl16]
                d1v = den1q_v[sl16]
                d0v = jnp.where(d0v > 0.0, d0v, 1.0)
                d1v = jnp.where(d1v > 0.0, d1v, 1.0)
                rv16 = d1v / d0v
                for t in range(16):
                    r = g * 16 + t
                    rb = jnp.full((16,), rv16[t], jnp.float32)
                    for j in range(HALF // 16):
                        sl = pl.ds(j * 16, 16)
                        sbuf_v[r, sl] = sbuf_v[r, sl] * rb
                return 0

            lax.fori_loop(0, 8, inner, 0)
            pltpu.sync_copy(frows, a_sh.at[pl.ds(l0, CHUNK)])
            return 0

        def p3b(q, _):
            l0 = row0 - lo + q * CHUNK
            rr = s * ROWQ + q
            frows = sbuf_v.at[pl.ds(0, CHUNK)]
            pltpu.sync_copy(a_sh.at[pl.ds(l0, CHUNK)], frows)
            pltpu.sync_copy(den1_sh.at[rr], den1q_v)
            pltpu.sync_copy(deg_sh.at[rr], degq_v)

            def inner(g, _):
                sl16 = pl.ds(pl.multiple_of(g * 16, 16), 16)
                d1v = den1q_v[sl16]
                d1v = jnp.where(d1v > 0.0, d1v, 1.0)
                inv16 = 1.0 / d1v
                nodev = row0 + q * CHUNK + g * 16 + lax.iota(jnp.int32, 16)
                w16 = jnp.where(nodev < N_NODES, degq_v[sl16], 0.0)
                for t in range(16):
                    r = g * 16 + t
                    invb = jnp.full((16,), inv16[t], jnp.float32)
                    wb = jnp.full((16,), w16[t], jnp.float32)
                    for j in range(HALF // 16):
                        sl = pl.ds(j * 16, 16)
                        hv = sbuf_v[r, sl] * invb
                        hv = jnp.where(hv > 0.0, hv, jnp.exp(hv) - 1.0)
                        hgq_v[sl] = hgq_v[sl] + wb * hv
                return 0

            lax.fori_loop(0, 8, inner, 0)
            return 0

        for i in range(3):
            cid = 3 * s + i

            @pl.when(cid < NHALF // CHUNK)
            def _():
                pltpu.sync_copy(z, a_sh.at[pl.ds(cid * CHUNK, CHUNK)])
        for j in range(HALF // 16):
            hgq_v[pl.ds(j * 16, 16)] = jnp.zeros((16,), jnp.float32)
        plsc.subcore_barrier()
        phase1(es0, ed0, ss0, sd0, False)
        phase2(tab0)
        plsc.subcore_barrier()
        own = (s // 8) == p

        @pl.when(own)
        def _():
            lax.fori_loop(0, ROWQ, p3a, 0)

        plsc.subcore_barrier()
        phase1(es1, ed1, ss1, sd1, False)
        phase2(tab1)
        plsc.subcore_barrier()

        @pl.when(own)
        def _():
            lax.fori_loop(0, ROWQ, p3b, 0)

        wid = p * (2 * N_TILES) + c * N_TILES + s
        pltpu.sync_copy(hgq_v, hg_out.at[wid])
        plsc.subcore_barrier()
        return 0

    lax.fori_loop(0, 2, fpass, 0)


def _sc_call(tab0, tab1, ss0, sd0, ss1, sd1, es0, ed0, es1, ed1, z):
    mesh = plsc.VectorSubcoreMesh(core_axis_name="c", subcore_axis_name="s")
    f32 = jnp.float32
    i32 = jnp.int32
    kern = pl.kernel(
        _sc_body,
        out_type=jax.ShapeDtypeStruct((4 * N_TILES, HALF), f32),
        mesh=mesh,
        compiler_params=pltpu.CompilerParams(needs_layout_passes=False),
        scratch_types=[
            pltpu.VMEM((2 * NROWS, CHUNK), f32),
            pltpu.VMEM((CHUNK, HALF), f32),
            pltpu.VMEM((CHUNKS, CHUNK), i32),
            pltpu.VMEM((CHUNKS, CHUNK), i32),
            pltpu.VMEM((CHUNKS, CHUNK), f32),
            pltpu.VMEM((NROWS, CHUNK), f32),
            pltpu.VMEM((CHUNK,), i32),
            pltpu.VMEM((CHUNK,), i32),
            pltpu.VMEM((CHUNK,), f32),
            pltpu.VMEM((CHUNK,), f32),
            pltpu.VMEM((CHUNK,), f32),
            pltpu.VMEM((HALF,), f32),
            pltpu.VMEM((NROWS,), i32),
            pltpu.SemaphoreType.DMA,
            pltpu.SemaphoreType.DMA,
            pltpu.SemaphoreType.DMA,
            pltpu.SemaphoreType.DMA,
            pltpu.VMEM_SHARED((AROWS, HALF), f32),
            pltpu.VMEM_SHARED((NROWS, CHUNK), f32),
            pltpu.VMEM_SHARED((NROWS, CHUNK), f32),
            pltpu.VMEM_SHARED((NROWS, CHUNK), f32),
        ],
    )
    return kern(tab0, tab1, ss0, sd0, ss1, sd1, es0, ed0, es1, ed1, z)


def _fin_body(parts_ref, wout_ref, bout_ref, out_ref):
    p = parts_ref[...]
    lo = jnp.sum(p[:N_TILES], axis=0) + jnp.sum(p[2 * N_TILES:3 * N_TILES],
                                                axis=0)
    hi = jnp.sum(p[N_TILES:2 * N_TILES], axis=0) + jnp.sum(p[3 * N_TILES:],
                                                           axis=0)
    hg = jnp.concatenate([lo, hi]) * jnp.float32(1.0 / N_NODES)
    r = jnp.dot(hg[None, :], wout_ref[...], preferred_element_type=jnp.float32)
    out_ref[...] = jax.nn.sigmoid(r + bout_ref[...])


def _final(parts, Wout, boutr):
    return pl.pallas_call(
        _fin_body,
        out_shape=jax.ShapeDtypeStruct((1, OUT_DIM), jnp.float32),
    )(parts, Wout, boutr)


def kernel(x, edge_index_rel0, edge_index_rel1, W0, b0, a0, W1, b1, a1,
           Wout, bout):
    f32 = jnp.float32
    xp = jnp.pad(x, ((0, NPAD - N_NODES), (0, 0)))
    b0r = b0.reshape(1, HID)
    b1r = b1.reshape(1, HID)
    a0r = a0[:, 0].reshape(2, HID)
    a1r = a1[:, 0].reshape(2, HID)

    wh0, wh1, s0, s1 = _matmuls(xp, W0, b0r, a0r, W1, b1r, a1r)
    tab0 = wh0.reshape(2 * NPAD, HALF)
    tab1 = wh1.reshape(2 * NPAD, HALF)
    ss0 = s0[:, 0, :].reshape(NROWS, CHUNK)
    sd0 = s0[:, 1, :].reshape(NROWS, CHUNK)
    ss1 = s1[:, 0, :].reshape(NROWS, CHUNK)
    sd1 = s1[:, 1, :].reshape(NROWS, CHUNK)

    pad = jnp.full((EPAD - N_EDGES,), N_NODES, jnp.int32)
    es0 = jnp.concatenate([edge_index_rel0[0], pad]).reshape(N_TILES, CHUNKS, CHUNK)
    ed0 = jnp.concatenate([edge_index_rel0[1], pad]).reshape(N_TILES, CHUNKS, CHUNK)
    es1 = jnp.concatenate([edge_index_rel1[0], pad]).reshape(N_TILES, CHUNKS, CHUNK)
    ed1 = jnp.concatenate([edge_index_rel1[1], pad]).reshape(N_TILES, CHUNKS, CHUNK)

    z = jnp.zeros((CHUNK, CHUNK), f32)

    parts = _sc_call(tab0, tab1, ss0, sd0, ss1, sd1, es0, ed0, es1, ed1, z)
    return _final(parts, Wout, bout.reshape(1, OUT_DIM))


# speedup vs baseline: 1.0533x; 1.0084x over previous
import jax
import jax.numpy as jnp
from jax import lax
from jax.experimental import pallas as pl
from jax.experimental.pallas import tpu as pltpu
from jax.experimental.pallas import tpu_sc as plsc

N_NODES = 10000
N_EDGES = 160000
NPAD = 10240
EPAD = 163840
IN_DIM = 256
HID = 256
HALF = 128
OUT_DIM = 64
N_TILES = 16
CHUNKS = 80
CHUNK = 128
NSEG = NPAD // N_TILES
ROWQ = NSEG // CHUNK
NROWS = NPAD // CHUNK
NHALF = NPAD // 2
TRASH = NHALF
AROWS = 5248


def _mm_body(x_ref, w0_ref, b0_ref, a0_ref, w1_ref, b1_ref, a1_ref,
             wh0_ref, wh1_ref, s0_ref, s1_ref):
    xb = x_ref[...]
    for w_ref, b_ref, a_ref, wh_ref, s_ref in (
            (w0_ref, b0_ref, a0_ref, wh0_ref, s0_ref),
            (w1_ref, b1_ref, a1_ref, wh1_ref, s1_ref)):
        wh = jnp.dot(xb, w_ref[...], preferred_element_type=jnp.float32)
        wh = wh + b_ref[...]
        wh_ref[...] = jnp.stack([wh[:, :HALF], wh[:, HALF:]], axis=0)
        sv = lax.dot_general(wh, a_ref[...], (((1,), (1,)), ((), ())),
                             preferred_element_type=jnp.float32)
        s_ref[...] = sv.T[None]


def _matmuls(xp, W0, b0r, a0r, W1, b1r, a1r):
    n_blk = NPAD // 256
    full = lambda shape: pl.BlockSpec(shape, lambda i: (0,) * len(shape))
    return pl.pallas_call(
        _mm_body,
        grid=(n_blk,),
        in_specs=[
            pl.BlockSpec((256, IN_DIM), lambda i: (i, 0)),
            full((IN_DIM, HID)), full((1, HID)), full((2, HID)),
            full((IN_DIM, HID)), full((1, HID)), full((2, HID)),
        ],
        out_specs=[
            pl.BlockSpec((2, 256, HALF), lambda i: (0, i, 0)),
            pl.BlockSpec((2, 256, HALF), lambda i: (0, i, 0)),
            pl.BlockSpec((1, 2, 256), lambda i: (i, 0, 0)),
            pl.BlockSpec((1, 2, 256), lambda i: (i, 0, 0)),
        ],
        out_shape=[
            jax.ShapeDtypeStruct((2, NPAD, HALF), jnp.float32),
            jax.ShapeDtypeStruct((2, NPAD, HALF), jnp.float32),
            jax.ShapeDtypeStruct((n_blk, 2, 256), jnp.float32),
            jax.ShapeDtypeStruct((n_blk, 2, 256), jnp.float32),
        ],
    )(xp, W0, b0r, a0r, W1, b1r, a1r)


def _sc_body(tab0, tab1, ss0, sd0, ss1, sd1, es0, ed0, es1, ed1, z,
             hg_out,
             sbuf_v, rowsb_v, src_v, dst_v, ex_v,
             dloc_v, dpa1_v, dpa2_v, dpb1_v, dpb2_v,
             den0q_v, den1q_v, degq_v, hgq_v, iota80_v,
             sema, semb, sema2, semb2,
             a_sh, den0_sh, den1_sh, deg_sh):
    c = lax.axis_index("c")
    s = lax.axis_index("s")
    row0 = s * NSEG

    for j in range(NROWS // 16):
        iota80_v[pl.ds(j * 16, 16)] = lax.iota(jnp.int32, 16) + j * 16

    pltpu.sync_copy(z.at[pl.ds(0, NROWS)], dloc_v)
    sl_seg = pl.ds(s * ROWQ, ROWQ)
    pltpu.sync_copy(dloc_v.at[sl_seg], den0_sh.at[sl_seg])
    pltpu.sync_copy(dloc_v.at[sl_seg], den1_sh.at[sl_seg])
    pltpu.sync_copy(dloc_v.at[sl_seg], deg_sh.at[sl_seg])
    plsc.subcore_barrier()

    def phase1(es_hbm, ed_hbm, ss_hbm, sd_hbm, histo):
        pltpu.sync_copy(ss_hbm, sbuf_v.at[pl.ds(0, NROWS)])
        pltpu.sync_copy(sd_hbm, sbuf_v.at[pl.ds(NROWS, NROWS)])
        pltpu.sync_copy(es_hbm.at[s], src_v)
        pltpu.sync_copy(ed_hbm.at[s], dst_v)

        def outer(k, _):
            for j in range(8):
                sl = pl.ds(j * 16, 16)
                sidx = src_v[k, sl]
                didx = dst_v[k, sl]
                shi = lax.shift_right_logical(sidx, 7)
                slo = jnp.bitwise_and(sidx, 127)
                dhi = lax.shift_right_logical(didx, 7)
                dlo = jnp.bitwise_and(didx, 127)
                sv = plsc.load_gather(sbuf_v, [shi, slo])
                dv = plsc.load_gather(sbuf_v, [dhi + NROWS, dlo])
                e = sv + dv
                e = jnp.where(e > 0.0, e, 0.01 * e)
                ex = jnp.exp(e)
                ex_v[k, sl] = ex
                if histo:
                    plsc.addupdate_scatter(dloc_v, [dhi, dlo], ex)
                src_v[k, sl] = sidx + c * NPAD
            return 0

        lax.fori_loop(0, CHUNKS, outer, 0)

    phase1(es0, ed0, ss0, sd0, True)
    pltpu.sync_copy(dloc_v, den0_sh.at[iota80_v], add=True)
    pltpu.sync_copy(z.at[pl.ds(0, NROWS)], dloc_v)
    phase1(es1, ed1, ss1, sd1, True)
    pltpu.sync_copy(dloc_v, den1_sh.at[iota80_v], add=True)
    pltpu.sync_copy(z.at[pl.ds(0, NROWS)], dloc_v)

    for es_hbm in (es0, es1):
        pltpu.sync_copy(es_hbm.at[s], src_v)

        def dcount(k, _):
            for j in range(8):
                sl = pl.ds(j * 16, 16)
                sidx = src_v[k, sl]
                shi = lax.shift_right_logical(sidx, 7)
                slo = jnp.bitwise_and(sidx, 127)
                plsc.addupdate_scatter(dloc_v, [shi, slo],
                                       jnp.ones((16,), jnp.float32))
            return 0

        lax.fori_loop(0, CHUNKS, dcount, 0)
    pltpu.sync_copy(dloc_v, deg_sh.at[iota80_v], add=True)
    plsc.subcore_barrier()

    def fpass(p, _):
        lo = p * NHALF

        bufa = sbuf_v.at[pl.ds(0, CHUNK)]
        a1 = sbuf_v.at[pl.ds(0, CHUNK // 2)]
        a2 = sbuf_v.at[pl.ds(CHUNK // 2, CHUNK // 2)]
        b1 = rowsb_v.at[pl.ds(0, CHUNK // 2)]
        b2 = rowsb_v.at[pl.ds(CHUNK // 2, CHUNK // 2)]

        def phase2(tab_hbm):
            def do_chunk(k, buf_dma, h1, h2, gsem, ssem, buf_ref, dp1, dp2):
                pltpu.make_async_copy(tab_hbm.at[src_v.at[k]], buf_dma,
                                      gsem).wait()
                for j in range(8):
                    sl = pl.ds(j * 16, 16)
                    dd = dst_v[k, sl] - lo
                    ok = jnp.logical_and(dd >= 0, dd < NHALF)
                    dp = dp1 if j < 4 else dp2
                    dp[pl.ds((j % 4) * 16, 16)] = jnp.where(ok, dd, TRASH)

                def inner(g, _):
                    exv16 = ex_v[k, pl.ds(pl.multiple_of(g * 16, 16), 16)]
                    for t in range(16):
                        r = g * 16 + t
                        exb = jnp.full((16,), exv16[t], jnp.float32)
                        for j in range(HALF // 16):
                            sl = pl.ds(j * 16, 16)
                            buf_ref[r, sl] = buf_ref[r, sl] * exb
                    return 0

                lax.fori_loop(0, 4, inner, 0)
                pltpu.async_copy(h1, a_sh.at[dp1], ssem, add=True)
                lax.fori_loop(4, 8, inner, 0)
                pltpu.async_copy(h2, a_sh.at[dp2], ssem, add=True)

            def drain2(h1, h2, dp1, dp2, ssem):
                pltpu.make_async_copy(h1, a_sh.at[dp1], ssem).wait()
                pltpu.make_async_copy(h2, a_sh.at[dp2], ssem).wait()

            pltpu.async_copy(tab_hbm.at[src_v.at[0]], bufa, sema)

            def outer(kk, _):
                k = 2 * kk
                pltpu.async_copy(tab_hbm.at[src_v.at[k + 1]], rowsb_v, semb)
                do_chunk(k, bufa, a1, a2, sema, sema2, sbuf_v,
                         dpa1_v, dpa2_v)

                @pl.when(kk < CHUNKS // 2 - 1)
                def _():
                    drain2(a1, a2, dpa1_v, dpa2_v, sema2)
                    pltpu.async_copy(tab_hbm.at[src_v.at[k + 2]], bufa, sema)

                do_chunk(k + 1, rowsb_v, b1, b2, semb, semb2, rowsb_v,
                         dpb1_v, dpb2_v)

                @pl.when(kk < CHUNKS // 2 - 1)
                def _():
                    drain2(b1, b2, dpb1_v, dpb2_v, semb2)

                return 0

            lax.fori_loop(0, CHUNKS // 2, outer, 0)
            drain2(a1, a2, dpa1_v, dpa2_v, sema2)
            drain2(b1, b2, dpb1_v, dpb2_v, semb2)

        def p3a(q, _):
            l0 = row0 - lo + q * CHUNK
            rr = s * ROWQ + q
            frows = sbuf_v.at[pl.ds(0, CHUNK)]
            pltpu.sync_copy(a_sh.at[pl.ds(l0, CHUNK)], frows)
            pltpu.sync_copy(den0_sh.at[rr], den0q_v)
            pltpu.sync_copy(den1_sh.at[rr], den1q_v)

            def inner(g, _):
                sl16 = pl.ds(pl.multiple_of(g * 16, 16), 16)
                d0v = den0q_v[sl16]
                d1v = den1q_v[sl16]
                d0v = jnp.where(d0v > 0.0, d0v, 1.0)
                d1v = jnp.where(d1v > 0.0, d1v, 1.0)
                rv16 = d1v / d0v
                for t in range(16):
                    r = g * 16 + t
                    rb = jnp.full((16,), rv16[t], jnp.float32)
                    for j in range(HALF // 16):
                        sl = pl.ds(j * 16, 16)
                        sbuf_v[r, sl] = sbuf_v[r, sl] * rb
                return 0

            lax.fori_loop(0, 8, inner, 0)
            pltpu.sync_copy(frows, a_sh.at[pl.ds(l0, CHUNK)])
            return 0

        def p3b(q, _):
            l0 = row0 - lo + q * CHUNK
            rr = s * ROWQ + q
            frows = sbuf_v.at[pl.ds(0, CHUNK)]
            pltpu.sync_copy(a_sh.at[pl.ds(l0, CHUNK)], frows)
            pltpu.sync_copy(den1_sh.at[rr], den1q_v)
            pltpu.sync_copy(deg_sh.at[rr], degq_v)

            def inner(g, _):
                sl16 = pl.ds(pl.multiple_of(g * 16, 16), 16)
                d1v = den1q_v[sl16]
                d1v = jnp.where(d1v > 0.0, d1v, 1.0)
                inv16 = 1.0 / d1v
                nodev = row0 + q * CHUNK + g * 16 + lax.iota(jnp.int32, 16)
                w16 = jnp.where(nodev < N_NODES, degq_v[sl16], 0.0)
                for t in range(16):
                    r = g * 16 + t
                    invb = jnp.full((16,), inv16[t], jnp.float32)
                    wb = jnp.full((16,), w16[t], jnp.float32)
                    for j in range(HALF // 16):
                        sl = pl.ds(j * 16, 16)
                        hv = sbuf_v[r, sl] * invb
                        hv = jnp.where(hv > 0.0, hv, jnp.exp(hv) - 1.0)
                        hgq_v[sl] = hgq_v[sl] + wb * hv
                return 0

            lax.fori_loop(0, 8, inner, 0)
            return 0

        for i in range(3):
            cid = 3 * s + i

            @pl.when(cid < NHALF // CHUNK)
            def _():
                pltpu.sync_copy(z, a_sh.at[pl.ds(cid * CHUNK, CHUNK)])
        for j in range(HALF // 16):
            hgq_v[pl.ds(j * 16, 16)] = jnp.zeros((16,), jnp.float32)
        plsc.subcore_barrier()
        phase1(es0, ed0, ss0, sd0, False)
        phase2(tab0)
        plsc.subcore_barrier()
        own = (s // 8) == p

        @pl.when(own)
        def _():
            lax.fori_loop(0, ROWQ, p3a, 0)

        plsc.subcore_barrier()
        phase1(es1, ed1, ss1, sd1, False)
        phase2(tab1)
        plsc.subcore_barrier()

        @pl.when(own)
        def _():
            lax.fori_loop(0, ROWQ, p3b, 0)

        wid = p * (2 * N_TILES) + c * N_TILES + s
        pltpu.sync_copy(hgq_v, hg_out.at[wid])
        plsc.subcore_barrier()
        return 0

    lax.fori_loop(0, 2, fpass, 0)


def _sc_call(tab0, tab1, ss0, sd0, ss1, sd1, es0, ed0, es1, ed1, z):
    mesh = plsc.VectorSubcoreMesh(core_axis_name="c", subcore_axis_name="s")
    f32 = jnp.float32
    i32 = jnp.int32
    kern = pl.kernel(
        _sc_body,
        out_type=jax.ShapeDtypeStruct((4 * N_TILES, HALF), f32),
        mesh=mesh,
        compiler_params=pltpu.CompilerParams(needs_layout_passes=False),
        scratch_types=[
            pltpu.VMEM((2 * NROWS, CHUNK), f32),
            pltpu.VMEM((CHUNK, HALF), f32),
            pltpu.VMEM((CHUNKS, CHUNK), i32),
            pltpu.VMEM((CHUNKS, CHUNK), i32),
            pltpu.VMEM((CHUNKS, CHUNK), f32),
            pltpu.VMEM((NROWS, CHUNK), f32),
            pltpu.VMEM((CHUNK // 2,), i32),
            pltpu.VMEM((CHUNK // 2,), i32),
            pltpu.VMEM((CHUNK // 2,), i32),
            pltpu.VMEM((CHUNK // 2,), i32),
            pltpu.VMEM((CHUNK,), f32),
            pltpu.VMEM((CHUNK,), f32),
            pltpu.VMEM((CHUNK,), f32),
            pltpu.VMEM((HALF,), f32),
            pltpu.VMEM((NROWS,), i32),
            pltpu.SemaphoreType.DMA,
            pltpu.SemaphoreType.DMA,
            pltpu.SemaphoreType.DMA,
            pltpu.SemaphoreType.DMA,
            pltpu.VMEM_SHARED((AROWS, HALF), f32),
            pltpu.VMEM_SHARED((NROWS, CHUNK), f32),
            pltpu.VMEM_SHARED((NROWS, CHUNK), f32),
            pltpu.VMEM_SHARED((NROWS, CHUNK), f32),
        ],
    )
    return kern(tab0, tab1, ss0, sd0, ss1, sd1, es0, ed0, es1, ed1, z)


def _fin_body(parts_ref, wout_ref, bout_ref, out_ref):
    p = parts_ref[...]
    lo = jnp.sum(p[:N_TILES], axis=0) + jnp.sum(p[2 * N_TILES:3 * N_TILES],
                                                axis=0)
    hi = jnp.sum(p[N_TILES:2 * N_TILES], axis=0) + jnp.sum(p[3 * N_TILES:],
                                                           axis=0)
    hg = jnp.concatenate([lo, hi]) * jnp.float32(1.0 / N_NODES)
    r = jnp.dot(hg[None, :], wout_ref[...], preferred_element_type=jnp.float32)
    out_ref[...] = jax.nn.sigmoid(r + bout_ref[...])


def _final(parts, Wout, boutr):
    return pl.pallas_call(
        _fin_body,
        out_shape=jax.ShapeDtypeStruct((1, OUT_DIM), jnp.float32),
    )(parts, Wout, boutr)


def kernel(x, edge_index_rel0, edge_index_rel1, W0, b0, a0, W1, b1, a1,
           Wout, bout):
    f32 = jnp.float32
    xp = jnp.pad(x, ((0, NPAD - N_NODES), (0, 0)))
    b0r = b0.reshape(1, HID)
    b1r = b1.reshape(1, HID)
    a0r = a0[:, 0].reshape(2, HID)
    a1r = a1[:, 0].reshape(2, HID)

    wh0, wh1, s0, s1 = _matmuls(xp, W0, b0r, a0r, W1, b1r, a1r)
    tab0 = wh0.reshape(2 * NPAD, HALF)
    tab1 = wh1.reshape(2 * NPAD, HALF)
    ss0 = s0[:, 0, :].reshape(NROWS, CHUNK)
    sd0 = s0[:, 1, :].reshape(NROWS, CHUNK)
    ss1 = s1[:, 0, :].reshape(NROWS, CHUNK)
    sd1 = s1[:, 1, :].reshape(NROWS, CHUNK)

    pad = jnp.full((EPAD - N_EDGES,), N_NODES, jnp.int32)
    es0 = jnp.concatenate([edge_index_rel0[0], pad]).reshape(N_TILES, CHUNKS, CHUNK)
    ed0 = jnp.concatenate([edge_index_rel0[1], pad]).reshape(N_TILES, CHUNKS, CHUNK)
    es1 = jnp.concatenate([edge_index_rel1[0], pad]).reshape(N_TILES, CHUNKS, CHUNK)
    ed1 = jnp.concatenate([edge_index_rel1[1], pad]).reshape(N_TILES, CHUNKS, CHUNK)

    z = jnp.zeros((CHUNK, CHUNK), f32)

    parts = _sc_call(tab0, tab1, ss0, sd0, ss1, sd1, es0, ed0, es1, ed1, z)
    return _final(parts, Wout, bout.reshape(1, OUT_DIM))
